# Initial kernel scaffold; baseline (speedup 1.0000x reference)
import jax
import jax.numpy as jnp
from jax.experimental import pallas as pl


def kernel(hidden_states, cos, sin, ln1_w, q_w, k_w, v_w, o_w, qn_w, kn_w, ln2_w, gate_w, w_gate, w_up, w_down):
    raise NotImplementedError("write your pallas kernel here")


# speedup vs baseline: 1.3365x; 1.3365x over previous
import jax
import jax.numpy as jnp
from jax.experimental import pallas as pl
from jax.experimental.pallas import tpu as pltpu

B, S, HID = 1, 2048, 2048
NH, NKV, HD = 16, 4, 128
E, TOPK, FF = 16, 2, 768
EPS = 1e-06
CAP = 512
REP = NH // NKV
NROWS = NH + 2 * NKV
BLK = 256
CBLK = 512

_f32 = jnp.float32
_bf16 = jnp.bfloat16


def _rms1_kernel(x_ref, ln_ref, h_ref):
    x = x_ref[...]
    inv = jax.lax.rsqrt(jnp.mean(x * x, axis=-1, keepdims=True) + EPS)
    h_ref[...] = (x * inv * ln_ref[...]).astype(_bf16)


def _qkv_head_kernel(h_ref, wq_ref, wk_ref, wv_ref, meta_ref, cos_ref, sin_ref, o_ref):
    j = pl.program_id(0)
    h = h_ref[...]
    wq = wq_ref[0]
    wk = wk_ref[0]
    wv = wv_ref[0]
    w = jnp.where(j < 8, wq, jnp.where(j < 10, wk, wv)).astype(_bf16)
    y = jax.lax.dot_general(h, w, (((1,), (1,)), ((), ())),
                            preferred_element_type=_f32)
    meta = meta_ref[0]
    cos = cos_ref[...]
    sin = sin_ref[...]
    for half in range(2):
        yh = y[:, half * HD:(half + 1) * HD]
        wrow = meta[half * 8 + 0:half * 8 + 1, :]
        nf = meta[half * 8 + 1:half * 8 + 2, :]
        post = meta[half * 8 + 2:half * 8 + 3, :]
        ss = jnp.mean(yh * yh, axis=-1, keepdims=True)
        scale = jax.lax.rsqrt(ss + EPS) * nf + (1.0 - nf)
        yh = yh * scale * wrow
        c = cos * nf + (1.0 - nf)
        s = sin * nf
        rot = jnp.concatenate([-yh[:, HD // 2:], yh[:, :HD // 2]], axis=-1)
        o_ref[half] = ((yh * c + rot * s) * post).astype(_bf16)


def _attn_kernel(q_ref, k_ref, v_ref, o_ref):
    q = q_ref[0]
    k = k_ref[0]
    v = v_ref[0]
    scores = jax.lax.dot_general(q, k, (((1,), (1,)), ((), ())),
                                 preferred_element_type=_f32)
    scores = scores * _f32(HD ** -0.5)
    row = jax.lax.broadcasted_iota(jnp.int32, (S, S), 0)
    col = jax.lax.broadcasted_iota(jnp.int32, (S, S), 1)
    scores = jnp.where(col <= row, scores, _f32(-1e30))
    m = jnp.max(scores, axis=-1, keepdims=True)
    p = jnp.exp(scores - m)
    denom = jnp.sum(p, axis=-1, keepdims=True)
    pb = (p / denom).astype(_bf16)
    o_ref[...] = (jnp.dot(pb, v, preferred_element_type=_f32)).astype(_bf16)


def _oproj_kernel(a_ref, w_ref, x_ref, o_ref):
    a = a_ref[...]
    o_ref[...] = x_ref[...] + jnp.dot(a, w_ref[...].astype(_bf16),
                                      preferred_element_type=_f32)


def _router_kernel(x_ref, ln_ref, gw_ref, h2_ref, route_ref, carry_ref):
    i = pl.program_id(0)

    @pl.when(i == 0)
    def _():
        carry_ref[...] = jnp.zeros_like(carry_ref)

    x = x_ref[...]
    inv = jax.lax.rsqrt(jnp.mean(x * x, axis=-1, keepdims=True) + EPS)
    h = x * inv * ln_ref[...]
    h2_ref[...] = h.astype(_bf16)
    logits = jax.lax.dot_general(
        h.astype(_bf16), gw_ref[...].astype(_bf16), (((1,), (0,)), ((), ())),
        preferred_element_type=_f32)
    lane = jax.lax.broadcasted_iota(jnp.int32, (BLK, 128), 1)
    neg = _f32(-1e30)
    logits = jnp.where(lane < E, logits, neg)
    m1 = jnp.max(logits, axis=-1, keepdims=True)
    i1 = jnp.min(jnp.where(logits == m1, lane, 10 ** 6), axis=-1, keepdims=True)
    l2 = jnp.where(lane == i1, neg, logits)
    m2 = jnp.max(l2, axis=-1, keepdims=True)
    i2 = jnp.min(jnp.where(l2 == m2, lane, 10 ** 6), axis=-1, keepdims=True)
    w1 = jax.nn.sigmoid(m1 - m2)
    w2 = 1.0 - w1
    oh0 = (lane == i1).astype(_f32)
    oh1 = (lane == i2).astype(_f32)
    ohs = oh0 + oh1
    r = jax.lax.broadcasted_iota(jnp.int32, (BLK, BLK), 0)
    c = jax.lax.broadcasted_iota(jnp.int32, (BLK, BLK), 1)
    ltri = (c < r).astype(_f32)
    pe = jnp.dot(ltri, ohs, preferred_element_type=_f32) + carry_ref[0:1, :]
    carry_ref[0:1, :] = carry_ref[0:1, :] + jnp.sum(ohs, axis=0, keepdims=True)
    pos0 = jnp.sum(pe * oh0, axis=-1, keepdims=True)
    pos1 = jnp.sum(pe * oh1, axis=-1, keepdims=True)
    keep0 = (pos0 < CAP).astype(_f32)
    keep1 = (pos1 < CAP).astype(_f32)
    p0m = jnp.where(pos0 < CAP, pos0, _f32(10 ** 6))
    p1m = jnp.where(pos1 < CAP, pos1, _f32(10 ** 6))
    z = jnp.zeros_like(pos0)
    route_ref[...] = jnp.concatenate(
        [i1.astype(_f32), i2.astype(_f32), p0m, p1m, w1 * keep0, w2 * keep1, z, z],
        axis=-1)


def _ffn_kernel(h2_ref, route_ref, wg_ref, wu_ref, wd_ref, y_ref):
    e = pl.program_id(0)
    ef = e.astype(_f32)
    r = route_ref[...]
    e0 = jnp.transpose(r[:, 0:1])
    e1 = jnp.transpose(r[:, 1:2])
    p0 = jnp.transpose(r[:, 2:3])
    p1 = jnp.transpose(r[:, 3:4])
    ci = jax.lax.broadcasted_iota(jnp.int32, (CAP, S), 0).astype(_f32)
    d = (jnp.where(jnp.logical_and(ci == p0, e0 == ef), _f32(1), _f32(0))
         + jnp.where(jnp.logical_and(ci == p1, e1 == ef), _f32(1), _f32(0)))
    buf = jnp.dot(d.astype(_bf16), h2_ref[...], preferred_element_type=_f32)
    b = buf.astype(_bf16)
    g = jnp.dot(b, wg_ref[0], preferred_element_type=_f32)
    u = jnp.dot(b, wu_ref[0], preferred_element_type=_f32)
    act = (g * jax.nn.sigmoid(g)) * u
    y_ref[0] = jnp.dot(act.astype(_bf16), wd_ref[0],
                       preferred_element_type=_f32).astype(_bf16)


def _combine_kernel(y_ref, route_ref, x_ref, o_ref):
    r = route_ref[...]
    o_ref[...] = x_ref[...]
    ci = jax.lax.broadcasted_iota(jnp.int32, (CBLK, CAP), 1).astype(_f32)
    p0 = r[:, 2:3]
    p1 = r[:, 3:4]
    e0 = r[:, 0:1]
    e1 = r[:, 1:2]
    w0 = r[:, 4:5]
    w1 = r[:, 5:6]

    def body(e, _):
        ef = e.astype(_f32)
        me0 = (e0 == ef).astype(_f32)
        me1 = (e1 == ef).astype(_f32)
        cb = (jnp.where(ci == p0, me0, _f32(0))
              + jnp.where(ci == p1, me1, _f32(0))).astype(_bf16)
        contrib = jnp.dot(cb, y_ref[e], preferred_element_type=_f32)
        wvec = w0 * me0 + w1 * me1
        o_ref[...] += contrib * wvec
        return 0

    jax.lax.fori_loop(0, E, body, 0)


def kernel(hidden_states, cos, sin, ln1_w, q_w, k_w, v_w, o_w, qn_w, kn_w,
           ln2_w, gate_w, w_gate, w_up, w_down):
    x = hidden_states.reshape(S, HID)
    cos0 = cos.reshape(S, HD)
    sin0 = sin.reshape(S, HD)
    ln1 = ln1_w.reshape(1, HID)
    ln2 = ln2_w.reshape(1, HID)

    h = pl.pallas_call(
        _rms1_kernel,
        grid=(S // BLK,),
        in_specs=[
            pl.BlockSpec((BLK, HID), lambda i: (i, 0)),
            pl.BlockSpec((1, HID), lambda i: (0, 0)),
        ],
        out_specs=pl.BlockSpec((BLK, HID), lambda i: (i, 0)),
        out_shape=jax.ShapeDtypeStruct((S, HID), _bf16),
    )(x, ln1)

    ones = jnp.ones((HD,), _f32)
    zeros = jnp.zeros((HD,), _f32)
    rows = []
    for j2 in range(NROWS):
        if j2 < NH:
            rows.append(jnp.stack([qn_w, ones, ones] + [zeros] * 5))
        elif j2 < NH + NKV:
            rows.append(jnp.stack([kn_w, ones, ones] + [zeros] * 5))
        else:
            rows.append(jnp.stack([ones, zeros, ones] + [zeros] * 5))
    meta = jnp.stack(rows).reshape(NROWS // 2, 16, HD)

    wq3 = q_w.reshape(8, 2 * HD, HID)
    wk3 = k_w.reshape(2, 2 * HD, HID)
    wv3 = v_w.reshape(2, 2 * HD, HID)

    qkv = pl.pallas_call(
        _qkv_head_kernel,
        grid=(NROWS // 2,),
        in_specs=[
            pl.BlockSpec((S, HID), lambda j: (0, 0)),
            pl.BlockSpec((1, 2 * HD, HID), lambda j: (jnp.minimum(j, 7), 0, 0)),
            pl.BlockSpec((1, 2 * HD, HID),
                         lambda j: (jnp.clip(j - 8, 0, 1), 0, 0)),
            pl.BlockSpec((1, 2 * HD, HID),
                         lambda j: (jnp.clip(j - 10, 0, 1), 0, 0)),
            pl.BlockSpec((1, 16, HD), lambda j: (j, 0, 0)),
            pl.BlockSpec((S, HD), lambda j: (0, 0)),
            pl.BlockSpec((S, HD), lambda j: (0, 0)),
        ],
        out_specs=pl.BlockSpec((2, S, HD), lambda j: (j, 0, 0)),
        out_shape=jax.ShapeDtypeStruct((NROWS, S, HD), _bf16),
    )(h, wq3, wk3, wv3, meta, cos0, sin0)

    attn = pl.pallas_call(
        _attn_kernel,
        grid=(NH,),
        in_specs=[
            pl.BlockSpec((1, S, HD), lambda hh: (hh, 0, 0)),
            pl.BlockSpec((1, S, HD), lambda hh: (NH + hh // REP, 0, 0)),
            pl.BlockSpec((1, S, HD), lambda hh: (NH + NKV + hh // REP, 0, 0)),
        ],
        out_specs=pl.BlockSpec((S, HD), lambda hh: (0, hh)),
        out_shape=jax.ShapeDtypeStruct((S, NH * HD), _bf16),
    )(qkv, qkv, qkv)

    x2 = pl.pallas_call(
        _oproj_kernel,
        grid=(S // BLK,),
        in_specs=[
            pl.BlockSpec((BLK, NH * HD), lambda i: (i, 0)),
            pl.BlockSpec((NH * HD, HID), lambda i: (0, 0)),
            pl.BlockSpec((BLK, HID), lambda i: (i, 0)),
        ],
        out_specs=pl.BlockSpec((BLK, HID), lambda i: (i, 0)),
        out_shape=jax.ShapeDtypeStruct((S, HID), _f32),
    )(attn, o_w.T, x)

    gwp = jnp.zeros((HID, 128), _f32).at[:, :E].set(gate_w.T)
    h2, route = pl.pallas_call(
        _router_kernel,
        grid=(S // BLK,),
        in_specs=[
            pl.BlockSpec((BLK, HID), lambda i: (i, 0)),
            pl.BlockSpec((1, HID), lambda i: (0, 0)),
            pl.BlockSpec((HID, 128), lambda i: (0, 0)),
        ],
        out_specs=[
            pl.BlockSpec((BLK, HID), lambda i: (i, 0)),
            pl.BlockSpec((BLK, 8), lambda i: (i, 0)),
        ],
        out_shape=[
            jax.ShapeDtypeStruct((S, HID), _bf16),
            jax.ShapeDtypeStruct((S, 8), _f32),
        ],
        scratch_shapes=[pltpu.VMEM((8, 128), _f32)],
        compiler_params=pltpu.CompilerParams(
            dimension_semantics=("arbitrary",)),
    )(x2, ln2, gwp)

    y = pl.pallas_call(
        _ffn_kernel,
        grid=(E,),
        in_specs=[
            pl.BlockSpec((S, HID), lambda e: (0, 0)),
            pl.BlockSpec((S, 8), lambda e: (0, 0)),
            pl.BlockSpec((1, HID, FF), lambda e: (e, 0, 0)),
            pl.BlockSpec((1, HID, FF), lambda e: (e, 0, 0)),
            pl.BlockSpec((1, FF, HID), lambda e: (e, 0, 0)),
        ],
        out_specs=pl.BlockSpec((1, CAP, HID), lambda e: (e, 0, 0)),
        out_shape=jax.ShapeDtypeStruct((E, CAP, HID), _bf16),
    )(h2, route, w_gate.astype(_bf16), w_up.astype(_bf16),
      w_down.astype(_bf16))

    out = pl.pallas_call(
        _combine_kernel,
        grid=(S // CBLK,),
        in_specs=[
            pl.BlockSpec((E, CAP, HID), lambda i: (0, 0, 0)),
            pl.BlockSpec((CBLK, 8), lambda i: (i, 0)),
            pl.BlockSpec((CBLK, HID), lambda i: (i, 0)),
        ],
        out_specs=pl.BlockSpec((CBLK, HID), lambda i: (i, 0)),
        out_shape=jax.ShapeDtypeStruct((S, HID), _f32),
    )(y, route, x2)

    return out.reshape(B, S, HID)
